# baseline (speedup 1.0000x reference)
import functools

import jax
import jax.numpy as jnp
from jax import lax
from jax.experimental import pallas as pl
from jax.experimental.pallas import tpu as pltpu
from jax.experimental.pallas import tpu_sc as plsc

D = 64
SCALE = 8.0
V = 1000000
B = 4096
S = 200

W1 = 256
NFULL = V // W1
VTAIL = V - NFULL * W1


def _iota16():
    return lax.iota(jnp.int32, 16)


def _bcast16(v):
    return jnp.zeros((16,), jnp.int32) + v


@jax.jit
def _stage1(tbl_t, x_t):
    info = plsc.get_sparse_core_info()
    nw = info.num_cores * info.num_subcores
    mesh = plsc.VectorSubcoreMesh(core_axis_name="c", subcore_axis_name="s")
    n_xt = (S // 8) * (B // 128)

    @functools.partial(
        pl.kernel,
        mesh=mesh,
        out_type=(
            jax.ShapeDtypeStruct((V * D,), jnp.float32),
            jax.ShapeDtypeStruct((n_xt, 8, 128), jnp.int32),
        ),
        scratch_types=[
            pltpu.VMEM((D, W1), jnp.float32),
            pltpu.VMEM((D, W1), jnp.float32),
            pltpu.VMEM((W1 * D,), jnp.float32),
            pltpu.VMEM((W1 * D,), jnp.float32),
            pltpu.VMEM((D, VTAIL), jnp.float32),
            pltpu.VMEM((VTAIL * D,), jnp.float32),
            pltpu.VMEM((8, 128), jnp.int32),
            pltpu.SemaphoreType.DMA,
            pltpu.SemaphoreType.DMA,
            pltpu.SemaphoreType.DMA,
            pltpu.SemaphoreType.DMA,
            pltpu.SemaphoreType.DMA,
        ],
        compiler_params=pltpu.CompilerParams(needs_layout_passes=False),
    )
    def k1(tbl_hbm, x_hbm, tflat_hbm, idx_hbm, s0, s1, d0, d1, st_v, dt_v,
           xt_v, si0, si1, so0, so1, xsem):
        wid = lax.axis_index("s") * info.num_cores + lax.axis_index("c")

        def xbody(t, carry):
            tile = wid + t * nw
            st = tile // (B // 128)
            bc = tile % (B // 128)
            pltpu.async_copy(
                x_hbm.at[pl.ds(st * 8, 8), pl.ds(bc * 128, 128)], xt_v, xsem
            ).wait()
            pltpu.async_copy(xt_v, idx_hbm.at[tile], xsem).wait()
            return carry

        lax.fori_loop(0, n_xt // nw, xbody, 0)

        nb = NFULL // nw + jnp.where(wid < NFULL % nw, 1, 0)

        def blk_of(i):
            return wid + i * nw

        def start_in(i, sbuf, sem):
            pltpu.async_copy(
                tbl_hbm.at[:, pl.ds(blk_of(i) * W1, W1)], sbuf, sem)

        def wait_in(i, sbuf, sem):
            pltpu.make_async_copy(
                tbl_hbm.at[:, pl.ds(blk_of(i) * W1, W1)], sbuf, sem).wait()

        def start_out(i, dbuf, sem):
            pltpu.async_copy(
                dbuf, tflat_hbm.at[pl.ds(blk_of(i) * W1 * D, W1 * D)], sem)

        def wait_out(i, dbuf, sem):
            pltpu.make_async_copy(
                dbuf, tflat_hbm.at[pl.ds(blk_of(i) * W1 * D, W1 * D)],
                sem).wait()

        def transpose(sbuf, dbuf, width):
            def tbody(vl, carry):
                vb = _bcast16(vl)
                for fc in range(D // 16):
                    vals = plsc.load_gather(sbuf, [fc * 16 + _iota16(), vb])
                    dbuf[pl.ds(vl * D + fc * 16, 16)] = vals * SCALE
                return carry

            lax.fori_loop(0, width, tbody, 0, unroll=2)

        def stage_step(i, sbuf, dbuf, sem_in, sem_out):
            wait_in(i, sbuf, sem_in)

            @pl.when(i >= 2)
            def _():
                wait_out(i - 2, dbuf, sem_out)

            transpose(sbuf, dbuf, W1)
            start_out(i, dbuf, sem_out)

            @pl.when(i + 2 < nb)
            def _():
                start_in(i + 2, sbuf, sem_in)

        start_in(0, s0, si0)

        @pl.when(nb > 1)
        def _():
            start_in(1, s1, si1)

        def loop_body(i, carry):
            @pl.when(i < nb)
            def _():
                @pl.when(i % 2 == 0)
                def _():
                    stage_step(i, s0, d0, si0, so0)

                @pl.when(i % 2 == 1)
                def _():
                    stage_step(i, s1, d1, si1, so1)

            return carry

        lax.fori_loop(0, NFULL // nw + 1, loop_body, 0)

        wait_out(0, d0, so0)
        wait_out(0, d1, so1)

        @pl.when(wid == 2)
        def _():
            v0 = NFULL * W1
            pltpu.async_copy(
                tbl_hbm.at[:, pl.ds(v0, VTAIL)], st_v, si0).wait()

            def tbody(vl, carry):
                vb = _bcast16(vl)
                for fc in range(D // 16):
                    vals = plsc.load_gather(st_v, [fc * 16 + _iota16(), vb])
                    dt_v[pl.ds(vl * D + fc * 16, 16)] = vals * SCALE
                return carry

            lax.fori_loop(0, VTAIL, tbody, 0, unroll=2)
            pltpu.async_copy(
                dt_v, tflat_hbm.at[pl.ds(v0 * D, VTAIL * D)], si0).wait()

    return k1(tbl_t, x_t)


@jax.jit
def _stage2(tbl_lin, idx2d):
    info = plsc.get_sparse_core_info()
    nw = info.num_cores * info.num_subcores
    mesh = plsc.VectorSubcoreMesh(core_axis_name="c", subcore_axis_name="s")
    NBT = B // 128
    n_half = (S // 8) * NBT * 2
    per_w = n_half // nw

    @functools.partial(
        pl.kernel,
        mesh=mesh,
        out_type=jax.ShapeDtypeStruct((S, 8, NBT, 1024), jnp.float32),
        scratch_types=[
            pltpu.VMEM((4, 128), jnp.int32),
            pltpu.VMEM((4, 128), jnp.int32),
            pltpu.VMEM((512, D), jnp.float32),
            pltpu.VMEM((512, D), jnp.float32),
            pltpu.VMEM((8, 1, 1024), jnp.float32),
            pltpu.VMEM((8, 1, 1024), jnp.float32),
            pltpu.SemaphoreType.DMA,
            pltpu.SemaphoreType.DMA,
            pltpu.SemaphoreType.DMA,
            pltpu.SemaphoreType.DMA,
            pltpu.SemaphoreType.DMA,
            pltpu.SemaphoreType.DMA,
        ],
        compiler_params=pltpu.CompilerParams(
            use_tc_tiling_on_sc=False, needs_layout_passes=False),
    )
    def k2(tbl_hbm, idx_hbm, out_hbm, i0, i1, r0, r1, t0, t1,
           gi0, gi1, gs0, gs1, os0, os1):
        wid = lax.axis_index("s") * info.num_cores + lax.axis_index("c")

        def hu_of(j):
            return wid + j * nw

        def start_idx(j, ibuf, sem):
            pltpu.async_copy(idx_hbm.at[pl.ds(hu_of(j) * 4, 4)], ibuf, sem)

        def wait_idx(ibuf, sem):
            pltpu.make_async_copy(idx_hbm.at[pl.ds(0, 4)], ibuf, sem).wait()

        def start_gathers(ibuf, rbuf, sem):
            for q in range(4):
                pltpu.async_copy(
                    tbl_hbm.at[ibuf.at[q]],
                    rbuf.at[pl.ds(q * 128, 128)], sem)

        def drain_gathers(ibuf, rbuf, sem):
            for q in range(4):
                pltpu.make_async_copy(
                    tbl_hbm.at[ibuf.at[q]],
                    rbuf.at[pl.ds(q * 128, 128)], sem).wait()

        def wait_out(tbuf, sem):
            pltpu.make_async_copy(
                tbuf, out_hbm.at[0, pl.ds(0, 8), pl.ds(0, 1)], sem).wait()

        def write_block(j, rbuf, sub_l, tbuf, sem):
            hu = hu_of(j)
            g = hu // 2
            h = hu % 2
            st = g // NBT
            bc = g % NBT
            s = st * 8 + h * 4 + sub_l

            def fbody(ft, carry):
                def f2body(sub2, c2):
                    fb = _bcast16(ft * 8 + sub2)
                    for c in range(8):
                        rows = sub_l * 128 + c * 16 + _iota16()
                        vals = plsc.load_gather(rbuf, [rows, fb])
                        tbuf[ft, 0, pl.ds(sub2 * 128 + c * 16, 16)] = vals
                    return c2

                lax.fori_loop(0, 8, f2body, 0)
                return carry

            lax.fori_loop(0, 8, fbody, 0)
            pltpu.async_copy(
                tbuf, out_hbm.at[s, pl.ds(0, 8), pl.ds(bc, 1)], sem)

        def step(j, ib, ib_n, rb, rb_n, gib, gib_n, gsb, gsb_n):
            drain_gathers(ib, rb, gsb)

            @pl.when(j + 1 < per_w)
            def _():
                wait_idx(ib_n, gib_n)
                start_gathers(ib_n, rb_n, gsb_n)

            @pl.when(j + 2 < per_w)
            def _():
                start_idx(j + 2, ib, gib)

            for sub_l in range(4):
                tb, osem = (t0, os0) if sub_l % 2 == 0 else (t1, os1)

                @pl.when(jnp.logical_or(j > 0, sub_l >= 2))
                def _():
                    wait_out(tb, osem)

                write_block(j, rb, sub_l, tb, osem)

        start_idx(0, i0, gi0)
        wait_idx(i0, gi0)
        start_gathers(i0, r0, gs0)

        @pl.when(per_w > 1)
        def _():
            start_idx(1, i1, gi1)

        def loop_body(j, carry):
            @pl.when(j % 2 == 0)
            def _():
                step(j, i0, i1, r0, r1, gi0, gi1, gs0, gs1)

            @pl.when(j % 2 == 1)
            def _():
                step(j, i1, i0, r1, r0, gi1, gi0, gs1, gs0)

            return carry

        lax.fori_loop(0, per_w, loop_body, 0)

        wait_out(t0, os0)
        wait_out(t1, os1)

    return k2(tbl_lin, idx2d)


def kernel(x, table):
    x = x.astype(jnp.int32)
    tbl_t = table.T
    x_t = x.T
    tbl_flat, idx4 = _stage1(tbl_t, x_t)
    tbl_lin = tbl_flat.reshape(V, D)
    idx2d = idx4.reshape((S // 8) * (B // 128) * 8, 128)
    out4 = _stage2(tbl_lin, idx2d)
    out5 = out4.reshape(S, 8, B // 128, 8, 128)
    return out5.transpose(2, 4, 0, 1, 3).reshape(B, S, D)


# speedup vs baseline: 1.2704x; 1.2704x over previous
import functools

import jax
import jax.numpy as jnp
from jax import lax
from jax.experimental import pallas as pl
from jax.experimental.pallas import tpu as pltpu
from jax.experimental.pallas import tpu_sc as plsc

D = 64
SCALE = 8.0
V = 1000000
B = 4096
S = 200

W1 = 256
NFULL = V // W1
VTAIL = V - NFULL * W1

NBT = B // 128
NST = S // 8


def _iota16():
    return lax.iota(jnp.int32, 16)


def _bcast16(v):
    return jnp.zeros((16,), jnp.int32) + v


@jax.jit
def _stage1(tbl_t, x_t):
    info = plsc.get_sparse_core_info()
    nw = info.num_cores * info.num_subcores
    mesh = plsc.VectorSubcoreMesh(core_axis_name="c", subcore_axis_name="s")
    n_xt = NST * NBT
    n_sup = n_xt // 4

    @functools.partial(
        pl.kernel,
        mesh=mesh,
        out_type=(
            jax.ShapeDtypeStruct((V * D,), jnp.float32),
            jax.ShapeDtypeStruct((n_xt, 8, 128), jnp.int32),
        ),
        scratch_types=[
            pltpu.VMEM((D, W1), jnp.float32),
            pltpu.VMEM((D, W1), jnp.float32),
            pltpu.VMEM((D * W1,), jnp.float32),
            pltpu.VMEM((D * W1,), jnp.float32),
            pltpu.VMEM((D, VTAIL), jnp.float32),
            pltpu.VMEM((D * VTAIL,), jnp.float32),
            pltpu.VMEM((8, 512), jnp.int32),
            pltpu.VMEM((8, 512), jnp.int32),
            pltpu.SemaphoreType.DMA,
            pltpu.SemaphoreType.DMA,
            pltpu.SemaphoreType.DMA,
            pltpu.SemaphoreType.DMA,
            pltpu.SemaphoreType.DMA,
            pltpu.SemaphoreType.DMA,
        ],
        compiler_params=pltpu.CompilerParams(needs_layout_passes=False),
    )
    def k1(tbl_hbm, x_hbm, tflat_hbm, idx_hbm, s0, s1, d0, d1, st_v, dt_v,
           xa, xb, si0, si1, so0, so1, xsem, wsem):
        wid = lax.axis_index("s") * info.num_cores + lax.axis_index("c")

        nb = NFULL // nw + jnp.where(wid < NFULL % nw, 1, 0)

        def blk_of(i):
            return wid + i * nw

        def start_in(i, sbuf, sem):
            pltpu.async_copy(
                tbl_hbm.at[:, pl.ds(blk_of(i) * W1, W1)], sbuf, sem)

        def wait_in(sbuf, sem):
            pltpu.make_async_copy(
                tbl_hbm.at[:, pl.ds(0, W1)], sbuf, sem).wait()

        def start_out(i, dbuf, sem):
            pltpu.async_copy(
                dbuf, tflat_hbm.at[pl.ds(blk_of(i) * W1 * D, W1 * D)], sem)

        def wait_out(dbuf, sem):
            pltpu.make_async_copy(
                dbuf, tflat_hbm.at[pl.ds(0, W1 * D)], sem).wait()

        start_in(0, s0, si0)

        @pl.when(nb > 1)
        def _():
            start_in(1, s1, si1)

        n_x = n_sup // nw + jnp.where(wid < n_sup % nw, 1, 0)

        def sup_of(t):
            return wid + t * nw

        def x_read(t, buf):
            sup = sup_of(t)
            st = sup // (NBT // 4)
            bc4 = sup % (NBT // 4)
            pltpu.async_copy(
                x_hbm.at[pl.ds(st * 8, 8), pl.ds(bc4 * 512, 512)], buf, xsem)

        def x_wait_read(buf):
            pltpu.make_async_copy(
                x_hbm.at[pl.ds(0, 8), pl.ds(0, 512)], buf, xsem).wait()

        def x_step(t, buf):
            x_wait_read(buf)
            sup = sup_of(t)
            tile0 = sup * 4
            for k in range(4):
                pltpu.async_copy(
                    buf.at[:, pl.ds(k * 128, 128)],
                    idx_hbm.at[tile0 + k], wsem).wait()

            @pl.when(t + 2 < n_x)
            def _():
                x_read(t + 2, buf)

        x_read(0, xa)

        @pl.when(n_x > 1)
        def _():
            x_read(1, xb)

        def x_loop(t, carry):
            @pl.when(t < n_x)
            def _():
                @pl.when(t % 2 == 0)
                def _():
                    x_step(t, xa)

                @pl.when(t % 2 == 1)
                def _():
                    x_step(t, xb)

            return carry

        lax.fori_loop(0, n_sup // nw + 1, x_loop, 0)

        iota = _iota16()
        rowc = tuple(fc * 16 + iota for fc in range(D // 16))

        def transpose(sbuf, dbuf):
            def tbody(vl, colv):
                vals = [plsc.load_gather(sbuf, [rowc[fc], colv])
                        for fc in range(D // 16)]
                off = vl * D
                for fc in range(D // 16):
                    dbuf[pl.ds(off + fc * 16, 16)] = vals[fc] * SCALE
                return colv + 1

            lax.fori_loop(0, W1, tbody, jnp.zeros((16,), jnp.int32),
                          unroll=4)

        def stage_step(i, sbuf, dbuf, sem_in, sem_out):
            wait_in(sbuf, sem_in)

            @pl.when(i >= 2)
            def _():
                wait_out(dbuf, sem_out)

            transpose(sbuf, dbuf)
            start_out(i, dbuf, sem_out)

            @pl.when(i + 2 < nb)
            def _():
                start_in(i + 2, sbuf, sem_in)

        def loop_body(i, carry):
            @pl.when(i < nb)
            def _():
                @pl.when(i % 2 == 0)
                def _():
                    stage_step(i, s0, d0, si0, so0)

                @pl.when(i % 2 == 1)
                def _():
                    stage_step(i, s1, d1, si1, so1)

            return carry

        lax.fori_loop(0, NFULL // nw + 1, loop_body, 0)

        wait_out(d0, so0)
        wait_out(d1, so1)

        @pl.when(wid == 2)
        def _():
            v0 = NFULL * W1
            pltpu.async_copy(
                tbl_hbm.at[:, pl.ds(v0, VTAIL)], st_v, si0).wait()

            def tbody(vl, colv):
                vals = [plsc.load_gather(st_v, [rowc[fc], colv])
                        for fc in range(D // 16)]
                off = vl * D
                for fc in range(D // 16):
                    dt_v[pl.ds(off + fc * 16, 16)] = vals[fc] * SCALE
                return colv + 1

            lax.fori_loop(0, VTAIL, tbody, jnp.zeros((16,), jnp.int32),
                          unroll=4)
            pltpu.async_copy(
                dt_v, tflat_hbm.at[pl.ds(v0 * D, VTAIL * D)], si0).wait()

    return k1(tbl_t, x_t)


@jax.jit
def _stage2(tbl_lin, idx2d):
    info = plsc.get_sparse_core_info()
    nw = info.num_cores * info.num_subcores
    mesh = plsc.VectorSubcoreMesh(core_axis_name="c", subcore_axis_name="s")
    n_half = NST * NBT * 2
    per_w = n_half // nw

    @functools.partial(
        pl.kernel,
        mesh=mesh,
        out_type=jax.ShapeDtypeStruct((S, 8, NBT, 1024), jnp.float32),
        scratch_types=[
            pltpu.VMEM((4, 128), jnp.int32),
            pltpu.VMEM((4, 128), jnp.int32),
            pltpu.VMEM((512, D), jnp.float32),
            pltpu.VMEM((512, D), jnp.float32),
            pltpu.VMEM((8, 1, 1024), jnp.float32),
            pltpu.VMEM((8, 1, 1024), jnp.float32),
            pltpu.SemaphoreType.DMA,
            pltpu.SemaphoreType.DMA,
            pltpu.SemaphoreType.DMA,
            pltpu.SemaphoreType.DMA,
            pltpu.SemaphoreType.DMA,
            pltpu.SemaphoreType.DMA,
        ],
        compiler_params=pltpu.CompilerParams(
            use_tc_tiling_on_sc=False, needs_layout_passes=False),
    )
    def k2(tbl_hbm, idx_hbm, out_hbm, i0, i1, r0, r1, t0, t1,
           gi0, gi1, gs0, gs1, os0, os1):
        wid = lax.axis_index("s") * info.num_cores + lax.axis_index("c")

        def hu_of(j):
            return wid + j * nw

        def start_idx(j, ibuf, sem):
            pltpu.async_copy(idx_hbm.at[pl.ds(hu_of(j) * 4, 4)], ibuf, sem)

        def wait_idx(ibuf, sem):
            pltpu.make_async_copy(idx_hbm.at[pl.ds(0, 4)], ibuf, sem).wait()

        def start_gathers(ibuf, rbuf, sem):
            for q in range(4):
                pltpu.async_copy(
                    tbl_hbm.at[ibuf.at[q]],
                    rbuf.at[pl.ds(q * 128, 128)], sem)

        def drain_gathers(ibuf, rbuf, sem):
            for q in range(4):
                pltpu.make_async_copy(
                    tbl_hbm.at[ibuf.at[q]],
                    rbuf.at[pl.ds(q * 128, 128)], sem).wait()

        def wait_out(tbuf, sem):
            pltpu.make_async_copy(
                tbuf, out_hbm.at[0, pl.ds(0, 8), pl.ds(0, 1)], sem).wait()

        iota = _iota16()

        def write_block(j, rbuf, sub_l, tbuf, sem):
            hu = hu_of(j)
            g = hu // 2
            h = hu % 2
            st = g // NBT
            bc = g % NBT
            s = st * 8 + h * 4 + sub_l
            row_base = [sub_l * 128 + c * 16 + iota for c in range(8)]

            def fbody(f, carry):
                fb = _bcast16(f)
                vals = [plsc.load_gather(rbuf, [row_base[c], fb])
                        for c in range(8)]
                ft = f // 8
                sub2 = f % 8
                off = sub2 * 128
                for c in range(8):
                    tbuf[ft, 0, pl.ds(off + c * 16, 16)] = vals[c]
                return carry

            lax.fori_loop(0, D, fbody, 0, unroll=2)
            pltpu.async_copy(
                tbuf, out_hbm.at[s, pl.ds(0, 8), pl.ds(bc, 1)], sem)

        def step(j, ib, ib_n, rb, rb_n, gib, gib_n, gsb, gsb_n):
            drain_gathers(ib, rb, gsb)

            @pl.when(j + 1 < per_w)
            def _():
                wait_idx(ib_n, gib_n)
                start_gathers(ib_n, rb_n, gsb_n)

            @pl.when(j + 2 < per_w)
            def _():
                start_idx(j + 2, ib, gib)

            for sub_l in range(4):
                tb, osem = (t0, os0) if sub_l % 2 == 0 else (t1, os1)

                @pl.when(jnp.logical_or(j > 0, sub_l >= 2))
                def _():
                    wait_out(tb, osem)

                write_block(j, rb, sub_l, tb, osem)

        start_idx(0, i0, gi0)
        wait_idx(i0, gi0)
        start_gathers(i0, r0, gs0)

        @pl.when(per_w > 1)
        def _():
            start_idx(1, i1, gi1)

        def loop_body(j, carry):
            @pl.when(j % 2 == 0)
            def _():
                step(j, i0, i1, r0, r1, gi0, gi1, gs0, gs1)

            @pl.when(j % 2 == 1)
            def _():
                step(j, i1, i0, r1, r0, gi1, gi0, gs1, gs0)

            return carry

        lax.fori_loop(0, per_w, loop_body, 0)

        wait_out(t0, os0)
        wait_out(t1, os1)

    return k2(tbl_lin, idx2d)


def kernel(x, table):
    x = x.astype(jnp.int32)
    tbl_t = table.T
    x_t = x.T
    tbl_flat, idx4 = _stage1(tbl_t, x_t)
    tbl_lin = tbl_flat.reshape(V, D)
    idx2d = idx4.reshape(NST * NBT * 8, 128)
    out4 = _stage2(tbl_lin, idx2d)
    out5 = out4.reshape(S, 8, NBT, 8, 128)
    return out5.transpose(2, 4, 0, 1, 3).reshape(B, S, D)
